# baseline (speedup 1.0000x reference)
import functools
import math

import jax
import jax.numpy as jnp
from jax import lax
from jax.experimental import pallas as pl
from jax.experimental.pallas import tpu as pltpu
from jax.experimental.pallas import tpu_sc as plsc

D_MODEL = 64
SCALE = math.sqrt(D_MODEL)

NC, NS, L = 2, 16, 16
NW = NC * NS

CHUNK = 128
NBG = 4
NBS = 2


@functools.lru_cache(maxsize=None)
def _make_kernel(n_pos: int, D: int, H: int):
    assert n_pos % NBG == 0 and n_pos // NBG >= 3
    mesh = plsc.VectorSubcoreMesh(core_axis_name="c", subcore_axis_name="s")

    @functools.partial(
        pl.kernel,
        mesh=mesh,
        out_type=jax.ShapeDtypeStruct((n_pos, D, NW * CHUNK), jnp.float32),
        scratch_types=[
            pltpu.VMEM((n_pos, CHUNK), jnp.int32),
            pltpu.VMEM((NBG, CHUNK), jnp.int32),
            pltpu.VMEM((NBG, CHUNK, 2 * D), jnp.float32),
            pltpu.VMEM((NBS, D, CHUNK), jnp.float32),
            pltpu.SemaphoreType.DMA,
            pltpu.SemaphoreType.DMA,
            pltpu.SemaphoreType.DMA,
            pltpu.SemaphoreType.DMA,
            pltpu.SemaphoreType.DMA,
            pltpu.SemaphoreType.DMA,
        ],
        compiler_params=pltpu.CompilerParams(needs_layout_passes=False),
    )
    def k(idx_hbm, pairs_hbm, out_hbm, idx_v, pring, gbuf, sbuf,
          gsem0, gsem1, gsem2, gsem3, ssem0, ssem1):
        gsem = (gsem0, gsem1, gsem2, gsem3)
        ssem = (ssem0, ssem1)
        wid = lax.axis_index("s") * NC + lax.axis_index("c")
        pltpu.sync_copy(idx_hbm.at[:, wid], idx_v)

        def prep_gather(j, b):
            @plsc.parallel_loop(0, CHUNK // L, unroll=8)
            def _(t0):
                iv = idx_v[j, pl.ds(t0 * L, L)]
                pring[b, pl.ds(t0 * L, L)] = jnp.where(iv >= H, iv - H, iv)
            pltpu.async_copy(pairs_hbm.at[pring.at[b]], gbuf.at[b], gsem[b])

        def gather_wait(b):
            pltpu.make_async_copy(
                pairs_hbm.at[pring.at[b]], gbuf.at[b], gsem[b]).wait()

        def store(j, sb):
            pltpu.async_copy(
                sbuf.at[sb], out_hbm.at[j, :, pl.ds(wid * CHUNK, CHUNK)],
                ssem[sb])

        def store_wait(sb):
            pltpu.make_async_copy(
                sbuf.at[sb], out_hbm.at[0, :, pl.ds(0, CHUNK)],
                ssem[sb]).wait()

        def extract(j, b, sb):
            tvecs = [lax.iota(jnp.int32, L) + (t0 * L)
                     for t0 in range(CHUNK // L)]
            hvecs = [jnp.where(idx_v[j, pl.ds(t0 * L, L)] >= H, D, 0)
                     for t0 in range(CHUNK // L)]

            @plsc.parallel_loop(0, D, unroll=16)
            def _(d):
                for t0 in range(CHUNK // L):
                    v = plsc.load_gather(gbuf.at[b],
                                         [tvecs[t0], hvecs[t0] + d])
                    sbuf[sb, d, pl.ds(t0 * L, L)] = v

        for j in range(NBG - 1):
            prep_gather(j, j)

        for j in range(NBG):
            gather_wait(j % NBG)
            prep_gather(j + NBG - 1, (j + NBG - 1) % NBG)
            if j >= NBS:
                store_wait(j % NBS)
            extract(j, j % NBG, j % NBS)
            store(j, j % NBS)

        def body(g, carry):
            for k4 in range(NBG):
                j = g * NBG + k4
                gather_wait(k4)
                prep_gather(j + NBG - 1, (k4 + NBG - 1) % NBG)
                store_wait(k4 % NBS)
                extract(j, k4, k4 % NBS)
                store(j, k4 % NBS)
            return carry

        lax.fori_loop(1, n_pos // NBG - 1, body, 0)

        for k4 in range(NBG):
            j = n_pos - NBG + k4
            gather_wait(k4)
            if k4 == 0:
                prep_gather(j + NBG - 1, (k4 + NBG - 1) % NBG)
            store_wait(k4 % NBS)
            extract(j, k4, k4 % NBS)
            store(j, k4 % NBS)

        for sb in range(NBS):
            store_wait(sb)

    return k


_PREP_BLK = 4096
_PAIR_OFF = 117 * _PREP_BLK
_PAIR_ROWS = 128 * _PREP_BLK


@functools.lru_cache(maxsize=None)
def _make_prep(V: int, D: int):
    assert _PAIR_OFF <= V <= _PAIR_ROWS + _PAIR_OFF

    def body(lo_ref, hi_ref, o_ref):
        o_ref[:, 0:D] = jnp.transpose(lo_ref[...], (1, 0)) * SCALE
        o_ref[:, D:2 * D] = jnp.transpose(hi_ref[...], (1, 0)) * SCALE

    return pl.pallas_call(
        body,
        grid=(_PAIR_ROWS // _PREP_BLK,),
        in_specs=[
            pl.BlockSpec((D, _PREP_BLK), lambda i: (0, i)),
            pl.BlockSpec((D, _PREP_BLK),
                         lambda i: (0, i + _PAIR_OFF // _PREP_BLK)),
        ],
        out_specs=pl.BlockSpec((_PREP_BLK, 2 * D), lambda i: (i, 0)),
        out_shape=jax.ShapeDtypeStruct((_PAIR_ROWS, 2 * D), jnp.float32),
    )


def kernel(x, table):
    S, T = x.shape
    V, D = table.shape
    idx = jnp.reshape(jnp.transpose(x).astype(jnp.int32), (T, NW, S // NW))
    tT = jnp.transpose(table)
    pairs = _make_prep(V, D)(tT, tT)
    out = _make_kernel(T, D, _PAIR_OFF)(idx, pairs)
    return jnp.transpose(out, (2, 0, 1))


# speedup vs baseline: 1.0081x; 1.0081x over previous
import functools
import math

import jax
import jax.numpy as jnp
from jax import lax
from jax.experimental import pallas as pl
from jax.experimental.pallas import tpu as pltpu
from jax.experimental.pallas import tpu_sc as plsc

D_MODEL = 64
SCALE = math.sqrt(D_MODEL)

NC, NS, L = 2, 16, 16
NW = NC * NS

CHUNK = 128
NBG = 4
NBS = 4


@functools.lru_cache(maxsize=None)
def _make_kernel(n_pos: int, D: int, H: int):
    assert n_pos % NBG == 0 and n_pos // NBG >= 3
    mesh = plsc.VectorSubcoreMesh(core_axis_name="c", subcore_axis_name="s")

    @functools.partial(
        pl.kernel,
        mesh=mesh,
        out_type=jax.ShapeDtypeStruct((n_pos, D, NW * CHUNK), jnp.float32),
        scratch_types=[
            pltpu.VMEM((n_pos, CHUNK), jnp.int32),
            pltpu.VMEM((NBG, CHUNK), jnp.int32),
            pltpu.VMEM((NBG, CHUNK, 2 * D), jnp.float32),
            pltpu.VMEM((NBS, D, CHUNK), jnp.float32),
            pltpu.SemaphoreType.DMA,
            pltpu.SemaphoreType.DMA,
            pltpu.SemaphoreType.DMA,
            pltpu.SemaphoreType.DMA,
            pltpu.SemaphoreType.DMA,
            pltpu.SemaphoreType.DMA,
            pltpu.SemaphoreType.DMA,
            pltpu.SemaphoreType.DMA,
        ],
        compiler_params=pltpu.CompilerParams(needs_layout_passes=False),
    )
    def k(idx_hbm, pairs_hbm, out_hbm, idx_v, pring, gbuf, sbuf,
          gsem0, gsem1, gsem2, gsem3, ssem0, ssem1, ssem2, ssem3):
        gsem = (gsem0, gsem1, gsem2, gsem3)
        ssem = (ssem0, ssem1, ssem2, ssem3)
        wid = lax.axis_index("s") * NC + lax.axis_index("c")
        pltpu.sync_copy(idx_hbm.at[:, wid], idx_v)

        def prep_gather(j, b):
            @plsc.parallel_loop(0, CHUNK // L, unroll=8)
            def _(t0):
                iv = idx_v[j, pl.ds(t0 * L, L)]
                pring[b, pl.ds(t0 * L, L)] = jnp.where(iv >= H, iv - H, iv)
            pltpu.async_copy(pairs_hbm.at[pring.at[b]], gbuf.at[b], gsem[b])

        def gather_wait(b):
            pltpu.make_async_copy(
                pairs_hbm.at[pring.at[b]], gbuf.at[b], gsem[b]).wait()

        def store(j, sb):
            pltpu.async_copy(
                sbuf.at[sb], out_hbm.at[j, :, pl.ds(wid * CHUNK, CHUNK)],
                ssem[sb])

        def store_wait(sb):
            pltpu.make_async_copy(
                sbuf.at[sb], out_hbm.at[0, :, pl.ds(0, CHUNK)],
                ssem[sb]).wait()

        def extract(j, b, sb):
            tvecs = [lax.iota(jnp.int32, L) + (t0 * L)
                     for t0 in range(CHUNK // L)]
            hvecs = [jnp.where(idx_v[j, pl.ds(t0 * L, L)] >= H, D, 0)
                     for t0 in range(CHUNK // L)]

            @plsc.parallel_loop(0, D, unroll=8)
            def _(d):
                for t0 in range(CHUNK // L):
                    v = plsc.load_gather(gbuf.at[b],
                                         [tvecs[t0], hvecs[t0] + d])
                    sbuf[sb, d, pl.ds(t0 * L, L)] = v

        for j in range(NBG - 1):
            prep_gather(j, j)

        for j in range(NBG):
            gather_wait(j % NBG)
            prep_gather(j + NBG - 1, (j + NBG - 1) % NBG)
            if j >= NBS:
                store_wait(j % NBS)
            extract(j, j % NBG, j % NBS)
            store(j, j % NBS)

        def body(g, carry):
            for k4 in range(NBG):
                j = g * NBG + k4
                gather_wait(k4)
                prep_gather(j + NBG - 1, (k4 + NBG - 1) % NBG)
                store_wait(k4 % NBS)
                extract(j, k4, k4 % NBS)
                store(j, k4 % NBS)
            return carry

        lax.fori_loop(1, n_pos // NBG - 1, body, 0)

        for k4 in range(NBG):
            j = n_pos - NBG + k4
            gather_wait(k4)
            if k4 == 0:
                prep_gather(j + NBG - 1, (k4 + NBG - 1) % NBG)
            store_wait(k4 % NBS)
            extract(j, k4, k4 % NBS)
            store(j, k4 % NBS)

        for sb in range(NBS):
            store_wait(sb)

    return k


_PREP_BLK = 4096
_PAIR_OFF = 117 * _PREP_BLK
_PAIR_ROWS = 128 * _PREP_BLK


@functools.lru_cache(maxsize=None)
def _make_prep(V: int, D: int):
    assert _PAIR_OFF <= V <= _PAIR_ROWS + _PAIR_OFF

    def body(lo_ref, hi_ref, o_ref):
        o_ref[:, 0:D] = jnp.transpose(lo_ref[...], (1, 0)) * SCALE
        o_ref[:, D:2 * D] = jnp.transpose(hi_ref[...], (1, 0)) * SCALE

    return pl.pallas_call(
        body,
        grid=(_PAIR_ROWS // _PREP_BLK,),
        in_specs=[
            pl.BlockSpec((D, _PREP_BLK), lambda i: (0, i)),
            pl.BlockSpec((D, _PREP_BLK),
                         lambda i: (0, i + _PAIR_OFF // _PREP_BLK)),
        ],
        out_specs=pl.BlockSpec((_PREP_BLK, 2 * D), lambda i: (i, 0)),
        out_shape=jax.ShapeDtypeStruct((_PAIR_ROWS, 2 * D), jnp.float32),
    )


def kernel(x, table):
    S, T = x.shape
    V, D = table.shape
    idx = jnp.reshape(jnp.transpose(x).astype(jnp.int32), (T, NW, S // NW))
    tT = jnp.transpose(table)
    pairs = _make_prep(V, D)(tT, tT)
    out = _make_kernel(T, D, _PAIR_OFF)(idx, pairs)
    return jnp.transpose(out, (2, 0, 1))
